# Initial kernel scaffold; baseline (speedup 1.0000x reference)
import jax
import jax.numpy as jnp
from jax.experimental import pallas as pl


def kernel(x, edge_index, W, b, ln_gamma, ln_beta):
    raise NotImplementedError("write your pallas kernel here")


# speedup vs baseline: 23.1132x; 23.1132x over previous
import functools

import jax
import jax.numpy as jnp
from jax import lax
from jax.experimental import pallas as pl
from jax.experimental.pallas import tpu as pltpu
from jax.experimental.pallas import tpu_sc as plsc

N = 10000
E = 320000
D = 128

NC, NS = 2, 16
NW = NC * NS
CHUNK = 64
NCHUNK = 160
EPT = NCHUNK * CHUNK
E_PAD = NW * EPT
ACC_ROWS = 10112
ZPT = ACC_ROWS // NS
Y_ROWS = N + 8

_mesh = plsc.VectorSubcoreMesh(core_axis_name="c", subcore_axis_name="s")


@functools.partial(
    pl.kernel,
    out_type=jax.ShapeDtypeStruct((NC * ACC_ROWS,), jnp.float32),
    mesh=_mesh,
    scratch_types=[
        pltpu.VMEM((NCHUNK, CHUNK), jnp.int32),
        pltpu.VMEM((CHUNK,), jnp.float32),
        pltpu.VMEM((ZPT + 8,), jnp.float32),
        pltpu.VMEM_SHARED((ACC_ROWS,), jnp.float32),
        pltpu.SemaphoreType.DMA,
    ],
)
def _deg_kernel(dst_hbm, deg_out, dst_v, ones_v, stage_v, degs, sem):
    cid = lax.axis_index("c")
    sid = lax.axis_index("s")
    w = sid * NC + cid
    row0 = sid * ZPT
    zeros16 = jnp.zeros((16,), jnp.float32)
    ones16 = jnp.ones((16,), jnp.float32)
    for j in range(CHUNK // 16):
        ones_v[pl.ds(j * 16, 16)] = ones16

    def zbody(j, carry):
        stage_v[pl.ds(j * 16, 16)] = zeros16
        return carry

    lax.fori_loop(0, (ZPT + 8) // 16, zbody, 0)
    pltpu.sync_copy(stage_v.at[pl.ds(0, ZPT)], degs.at[pl.ds(row0, ZPT)])
    pltpu.sync_copy(dst_hbm.at[w], dst_v)
    plsc.subcore_barrier()

    def body(c, carry):
        pltpu.async_copy(ones_v, degs.at[dst_v.at[c]], sem, add=True)
        pltpu.make_async_copy(ones_v, degs.at[dst_v.at[0]], sem).wait()
        return carry

    lax.fori_loop(0, NCHUNK, body, 0)
    plsc.subcore_barrier()
    pltpu.sync_copy(degs.at[pl.ds(row0, ZPT)], stage_v.at[pl.ds(0, ZPT)])
    pltpu.sync_copy(stage_v.at[pl.ds(0, ZPT)],
                    deg_out.at[pl.ds(cid * ACC_ROWS + row0, ZPT)])


@functools.partial(
    pl.kernel,
    out_type=jax.ShapeDtypeStruct((NC, ACC_ROWS, D), jnp.float32),
    mesh=_mesh,
    scratch_types=[
        pltpu.VMEM((CHUNK,), jnp.int32),
        pltpu.VMEM((CHUNK,), jnp.int32),
        pltpu.VMEM((CHUNK,), jnp.int32),
        pltpu.VMEM((CHUNK,), jnp.int32),
        pltpu.VMEM((CHUNK, D), jnp.float32),
        pltpu.VMEM((CHUNK, D), jnp.float32),
        pltpu.VMEM_SHARED((ACC_ROWS, D), jnp.float32),
        pltpu.SemaphoreType.DMA,
        pltpu.SemaphoreType.DMA,
        pltpu.SemaphoreType.DMA,
        pltpu.SemaphoreType.DMA,
        pltpu.SemaphoreType.DMA,
        pltpu.SemaphoreType.DMA,
    ],
)
def _edge_kernel(y_hbm, src_hbm, dst_hbm, out_hbm,
                 sidx0, sidx1, didx0, didx1, buf0, buf1, acc,
                 gsem0, gsem1, ssem0, ssem1, isem0, isem1):
    cid = lax.axis_index("c")
    sid = lax.axis_index("s")
    w = sid * NC + cid
    row0 = sid * ZPT
    zeros16 = jnp.zeros((16,), jnp.float32)

    def zbody(r, carry):
        for j in range(D // 16):
            buf0[r, pl.ds(j * 16, 16)] = zeros16
        return carry

    lax.fori_loop(0, CHUNK, zbody, 0)
    _rem = ZPT % CHUNK
    for k in range(ZPT // CHUNK):
        pltpu.sync_copy(buf0, acc.at[pl.ds(row0 + k * CHUNK, CHUNK)])
    if _rem:
        pltpu.sync_copy(buf0.at[pl.ds(0, _rem)],
                        acc.at[pl.ds(row0 + (ZPT // CHUNK) * CHUNK, _rem)])
    plsc.subcore_barrier()

    bufs = (buf0, buf1)
    sidx = (sidx0, sidx1)
    didx = (didx0, didx1)
    gsems = (gsem0, gsem1)
    ssems = (ssem0, ssem1)
    isems = (isem0, isem1)

    def start_fetch(c, b):
        pltpu.async_copy(src_hbm.at[w, c], sidx[b], isems[b])
        pltpu.async_copy(dst_hbm.at[w, c], didx[b], isems[b])

    def wait_fetch(b):
        pltpu.make_async_copy(src_hbm.at[0, 0], sidx[b], isems[b]).wait()
        pltpu.make_async_copy(dst_hbm.at[0, 0], didx[b], isems[b]).wait()

    def start_gather(b):
        pltpu.async_copy(y_hbm.at[sidx[b]], bufs[b], gsems[b])

    def wait_gather(b):
        pltpu.make_async_copy(y_hbm.at[sidx[b]], bufs[b], gsems[b]).wait()

    def start_scatter(b):
        pltpu.async_copy(bufs[b], acc.at[didx[b]], ssems[b], add=True)

    def wait_scatter(b):
        pltpu.make_async_copy(bufs[b], acc.at[didx[b]], ssems[b]).wait()

    start_fetch(0, 0)
    start_fetch(1, 1)
    wait_fetch(0)
    start_gather(0)

    def pair(g, carry):
        c0 = g * 2
        for b in range(2):
            c = c0 + b
            wait_gather(b)
            start_scatter(b)
            wait_scatter(b)
            wait_fetch(1 - b)
            start_fetch(c + 2, b)
            start_gather(1 - b)
        return carry

    lax.fori_loop(0, (NCHUNK - 2) // 2, pair, 0)
    wait_gather(0)
    start_scatter(0)
    wait_scatter(0)
    wait_fetch(1)
    start_gather(1)
    wait_gather(1)
    start_scatter(1)
    wait_scatter(1)
    plsc.subcore_barrier()
    for k in range(ZPT // CHUNK):
        pltpu.sync_copy(acc.at[pl.ds(row0 + k * CHUNK, CHUNK)], buf0)
        pltpu.sync_copy(buf0, out_hbm.at[cid, pl.ds(row0 + k * CHUNK, CHUNK)])
    if _rem:
        _off = row0 + (ZPT // CHUNK) * CHUNK
        pltpu.sync_copy(acc.at[pl.ds(_off, _rem)], buf0.at[pl.ds(0, _rem)])
        pltpu.sync_copy(buf0.at[pl.ds(0, _rem)],
                        out_hbm.at[cid, pl.ds(_off, _rem)])


_BLK = 1000


def _mm_body(x_ref, w_ref, d0_ref, d1_ref, y_ref, dinv_ref):
    deg = d0_ref[...] + d1_ref[...] + 1.0
    dinv = lax.rsqrt(deg)
    xw = jnp.dot(x_ref[...], w_ref[...], preferred_element_type=jnp.float32)
    y_ref[...] = xw * dinv
    dinv_ref[...] = dinv


def _fin_body(a0_ref, a1_ref, y_ref, dinv_ref, b_ref, g_ref, be_ref, o_ref):
    s = a0_ref[0] + a1_ref[0] + y_ref[...]
    pre = s * dinv_ref[...] + b_ref[...]
    mu = jnp.mean(pre, axis=-1, keepdims=True)
    ctr = pre - mu
    var = jnp.mean(ctr * ctr, axis=-1, keepdims=True)
    h = ctr * lax.rsqrt(var + 1e-5) * g_ref[...] + be_ref[...]
    o_ref[...] = jnp.maximum(h, 0.0)


def kernel(x, edge_index, W, b, ln_gamma, ln_beta):
    ei = edge_index.astype(jnp.int32)
    pad = N + (jnp.arange(E_PAD - E, dtype=jnp.int32) % 8)
    src_p = jnp.concatenate([ei[0], pad]).reshape(NW, NCHUNK, CHUNK)
    dst_p = jnp.concatenate([ei[1], pad]).reshape(NW, NCHUNK, CHUNK)

    deg_parts = _deg_kernel(dst_p)
    d0 = deg_parts[:N].reshape(N, 1)
    d1 = deg_parts[ACC_ROWS:ACC_ROWS + N].reshape(N, 1)

    y, dinv = pl.pallas_call(
        _mm_body,
        grid=(N // _BLK,),
        in_specs=[
            pl.BlockSpec((_BLK, D), lambda i: (i, 0)),
            pl.BlockSpec((D, D), lambda i: (0, 0)),
            pl.BlockSpec((_BLK, 1), lambda i: (i, 0)),
            pl.BlockSpec((_BLK, 1), lambda i: (i, 0)),
        ],
        out_specs=[
            pl.BlockSpec((_BLK, D), lambda i: (i, 0)),
            pl.BlockSpec((_BLK, 1), lambda i: (i, 0)),
        ],
        out_shape=[
            jax.ShapeDtypeStruct((N, D), jnp.float32),
            jax.ShapeDtypeStruct((N, 1), jnp.float32),
        ],
    )(x, W, d0, d1)

    y_pad = jnp.concatenate([y, jnp.zeros((Y_ROWS - N, D), jnp.float32)])
    acc_parts = _edge_kernel(y_pad, src_p, dst_p)

    out = pl.pallas_call(
        _fin_body,
        grid=(N // _BLK,),
        in_specs=[
            pl.BlockSpec((1, _BLK, D), lambda i: (0, i, 0)),
            pl.BlockSpec((1, _BLK, D), lambda i: (1, i, 0)),
            pl.BlockSpec((_BLK, D), lambda i: (i, 0)),
            pl.BlockSpec((_BLK, 1), lambda i: (i, 0)),
            pl.BlockSpec((1, D), lambda i: (0, 0)),
            pl.BlockSpec((1, D), lambda i: (0, 0)),
            pl.BlockSpec((1, D), lambda i: (0, 0)),
        ],
        out_specs=pl.BlockSpec((_BLK, D), lambda i: (i, 0)),
        out_shape=jax.ShapeDtypeStruct((N, D), jnp.float32),
    )(acc_parts, acc_parts, y, dinv,
      b.reshape(1, D), ln_gamma.reshape(1, D), ln_beta.reshape(1, D))
    return out


# speedup vs baseline: 25.1642x; 1.0887x over previous
import functools

import jax
import jax.numpy as jnp
from jax import lax
from jax.experimental import pallas as pl
from jax.experimental.pallas import tpu as pltpu
from jax.experimental.pallas import tpu_sc as plsc

N = 10000
E = 320000
D = 128

NC, NS = 2, 16
NW = NC * NS
CHUNK = 64
NCHUNK = 160
EPT = NCHUNK * CHUNK
E_PAD = NW * EPT
ACC_ROWS = 10112
ZPT = ACC_ROWS // NS
Y_ROWS = N + 8

_mesh = plsc.VectorSubcoreMesh(core_axis_name="c", subcore_axis_name="s")


@functools.partial(
    pl.kernel,
    out_type=jax.ShapeDtypeStruct((NC * ACC_ROWS,), jnp.float32),
    mesh=_mesh,
    scratch_types=[
        [pltpu.VMEM((CHUNK,), jnp.int32)] * 4,
        pltpu.VMEM((CHUNK,), jnp.float32),
        pltpu.VMEM((ZPT + 8,), jnp.float32),
        pltpu.VMEM_SHARED((ACC_ROWS,), jnp.float32),
        [pltpu.SemaphoreType.DMA] * 4,
        [pltpu.SemaphoreType.DMA] * 4,
    ],
)
def _deg_kernel(dst_hbm, deg_out, didx, ones_v, stage_v, degs, isems, ssems):
    cid = lax.axis_index("c")
    sid = lax.axis_index("s")
    w = sid * NC + cid
    row0 = sid * ZPT
    zeros16 = jnp.zeros((16,), jnp.float32)
    ones16 = jnp.ones((16,), jnp.float32)
    for j in range(CHUNK // 16):
        ones_v[pl.ds(j * 16, 16)] = ones16

    def zbody(j, carry):
        stage_v[pl.ds(j * 16, 16)] = zeros16
        return carry

    lax.fori_loop(0, (ZPT + 8) // 16, zbody, 0)
    pltpu.sync_copy(stage_v.at[pl.ds(0, ZPT)], degs.at[pl.ds(row0, ZPT)])
    plsc.subcore_barrier()

    def start_fetch(c, s):
        pltpu.async_copy(dst_hbm.at[w, c], didx[s], isems[s])

    def wait_fetch(s):
        pltpu.make_async_copy(dst_hbm.at[0, 0], didx[s], isems[s]).wait()

    def start_scatter(s):
        pltpu.async_copy(ones_v, degs.at[didx[s]], ssems[s], add=True)

    def wait_scatter(s):
        pltpu.make_async_copy(ones_v, degs.at[didx[s]], ssems[s]).wait()

    for s in range(4):
        start_fetch(s, s)
    for c in range(2):
        wait_fetch(c)
        start_scatter(c)

    def group(g, carry):
        c0 = g * 4 + 2
        for k in range(4):
            c = c0 + k
            b0 = (2 + k) % 4
            b2 = k % 4
            wait_fetch(b0)
            start_scatter(b0)
            wait_scatter(b2)
            start_fetch(c + 2, b2)
        return carry

    lax.fori_loop(0, (NCHUNK - 4) // 4, group, 0)
    for c in range(NCHUNK - 2, NCHUNK):
        wait_fetch(c % 4)
        start_scatter(c % 4)
    for s in range(4):
        wait_scatter(s)
    plsc.subcore_barrier()
    pltpu.sync_copy(degs.at[pl.ds(row0, ZPT)], stage_v.at[pl.ds(0, ZPT)])
    pltpu.sync_copy(stage_v.at[pl.ds(0, ZPT)],
                    deg_out.at[pl.ds(cid * ACC_ROWS + row0, ZPT)])


@functools.partial(
    pl.kernel,
    out_type=jax.ShapeDtypeStruct((NC, ACC_ROWS, D), jnp.float32),
    mesh=_mesh,
    scratch_types=[
        [pltpu.VMEM((CHUNK,), jnp.int32)] * 4,
        [pltpu.VMEM((CHUNK,), jnp.int32)] * 4,
        [pltpu.VMEM((CHUNK, D), jnp.float32)] * 4,
        pltpu.VMEM_SHARED((ACC_ROWS, D), jnp.float32),
        [pltpu.SemaphoreType.DMA] * 4,
        [pltpu.SemaphoreType.DMA] * 4,
        [pltpu.SemaphoreType.DMA] * 4,
    ],
)
def _edge_kernel(y_hbm, src_hbm, dst_hbm, out_hbm,
                 sidx, didx, bufs, acc, gsems, ssems, isems):
    cid = lax.axis_index("c")
    sid = lax.axis_index("s")
    w = sid * NC + cid
    row0 = sid * ZPT
    zeros16 = jnp.zeros((16,), jnp.float32)
    buf0 = bufs[0]

    def zbody(r, carry):
        for j in range(D // 16):
            buf0[r, pl.ds(j * 16, 16)] = zeros16
        return carry

    lax.fori_loop(0, CHUNK, zbody, 0)
    _rem = ZPT % CHUNK
    for k in range(ZPT // CHUNK):
        pltpu.sync_copy(buf0, acc.at[pl.ds(row0 + k * CHUNK, CHUNK)])
    if _rem:
        pltpu.sync_copy(buf0.at[pl.ds(0, _rem)],
                        acc.at[pl.ds(row0 + (ZPT // CHUNK) * CHUNK, _rem)])
    plsc.subcore_barrier()

    def start_fetch(c, s):
        pltpu.async_copy(src_hbm.at[w, c], sidx[s], isems[s])
        pltpu.async_copy(dst_hbm.at[w, c], didx[s], isems[s])

    def wait_fetch(s):
        pltpu.make_async_copy(src_hbm.at[0, 0], sidx[s], isems[s]).wait()
        pltpu.make_async_copy(dst_hbm.at[0, 0], didx[s], isems[s]).wait()

    def start_gather(s):
        pltpu.async_copy(y_hbm.at[sidx[s]], bufs[s], gsems[s])

    def wait_gather(s):
        pltpu.make_async_copy(y_hbm.at[sidx[s]], bufs[s], gsems[s]).wait()

    def start_scatter(s):
        pltpu.async_copy(bufs[s], acc.at[didx[s]], ssems[s], add=True)

    def wait_scatter(s):
        pltpu.make_async_copy(bufs[s], acc.at[didx[s]], ssems[s]).wait()

    for s in range(4):
        start_fetch(s, s)
    wait_fetch(0)
    start_gather(0)
    for c in range(2):
        wait_gather(c)
        start_scatter(c)
        wait_fetch(c + 1)
        start_gather(c + 1)

    def group(g, carry):
        c0 = g * 4 + 2
        for k in range(4):
            c = c0 + k
            b0 = (2 + k) % 4
            b1 = (3 + k) % 4
            b2 = k % 4
            wait_gather(b0)
            start_scatter(b0)
            wait_scatter(b2)
            start_fetch(c + 2, b2)
            wait_fetch(b1)
            start_gather(b1)
        return carry

    lax.fori_loop(0, (NCHUNK - 4) // 4, group, 0)
    wait_gather(2)
    start_scatter(2)
    wait_scatter(0)
    wait_fetch(3)
    start_gather(3)
    wait_gather(3)
    start_scatter(3)
    wait_scatter(1)
    wait_scatter(2)
    wait_scatter(3)
    plsc.subcore_barrier()
    for k in range(ZPT // CHUNK):
        pltpu.sync_copy(acc.at[pl.ds(row0 + k * CHUNK, CHUNK)], buf0)
        pltpu.sync_copy(buf0, out_hbm.at[cid, pl.ds(row0 + k * CHUNK, CHUNK)])
    if _rem:
        _off = row0 + (ZPT // CHUNK) * CHUNK
        pltpu.sync_copy(acc.at[pl.ds(_off, _rem)], buf0.at[pl.ds(0, _rem)])
        pltpu.sync_copy(buf0.at[pl.ds(0, _rem)],
                        out_hbm.at[cid, pl.ds(_off, _rem)])


_BLK = 1000


def _mm_body(x_ref, w_ref, d0_ref, d1_ref, y_ref, dinv_ref):
    deg = d0_ref[...] + d1_ref[...] + 1.0
    dinv = lax.rsqrt(deg)
    xw = jnp.dot(x_ref[...], w_ref[...], preferred_element_type=jnp.float32)
    y_ref[...] = xw * dinv
    dinv_ref[...] = dinv


def _fin_body(a0_ref, a1_ref, y_ref, dinv_ref, b_ref, g_ref, be_ref, o_ref):
    s = a0_ref[0] + a1_ref[0] + y_ref[...]
    pre = s * dinv_ref[...] + b_ref[...]
    mu = jnp.mean(pre, axis=-1, keepdims=True)
    ctr = pre - mu
    var = jnp.mean(ctr * ctr, axis=-1, keepdims=True)
    h = ctr * lax.rsqrt(var + 1e-5) * g_ref[...] + be_ref[...]
    o_ref[...] = jnp.maximum(h, 0.0)


def kernel(x, edge_index, W, b, ln_gamma, ln_beta):
    ei = edge_index.astype(jnp.int32)
    pad = N + (jnp.arange(E_PAD - E, dtype=jnp.int32) % 8)
    src_p = jnp.concatenate([ei[0], pad]).reshape(NW, NCHUNK, CHUNK)
    dst_p = jnp.concatenate([ei[1], pad]).reshape(NW, NCHUNK, CHUNK)

    deg_parts = _deg_kernel(dst_p)
    d0 = deg_parts[:N].reshape(N, 1)
    d1 = deg_parts[ACC_ROWS:ACC_ROWS + N].reshape(N, 1)

    y, dinv = pl.pallas_call(
        _mm_body,
        grid=(N // _BLK,),
        in_specs=[
            pl.BlockSpec((_BLK, D), lambda i: (i, 0)),
            pl.BlockSpec((D, D), lambda i: (0, 0)),
            pl.BlockSpec((_BLK, 1), lambda i: (i, 0)),
            pl.BlockSpec((_BLK, 1), lambda i: (i, 0)),
        ],
        out_specs=[
            pl.BlockSpec((_BLK, D), lambda i: (i, 0)),
            pl.BlockSpec((_BLK, 1), lambda i: (i, 0)),
        ],
        out_shape=[
            jax.ShapeDtypeStruct((N, D), jnp.float32),
            jax.ShapeDtypeStruct((N, 1), jnp.float32),
        ],
    )(x, W, d0, d1)

    y_pad = jnp.concatenate([y, jnp.zeros((Y_ROWS - N, D), jnp.float32)])
    acc_parts = _edge_kernel(y_pad, src_p, dst_p)

    out = pl.pallas_call(
        _fin_body,
        grid=(N // _BLK,),
        in_specs=[
            pl.BlockSpec((1, _BLK, D), lambda i: (0, i, 0)),
            pl.BlockSpec((1, _BLK, D), lambda i: (1, i, 0)),
            pl.BlockSpec((_BLK, D), lambda i: (i, 0)),
            pl.BlockSpec((_BLK, 1), lambda i: (i, 0)),
            pl.BlockSpec((1, D), lambda i: (0, 0)),
            pl.BlockSpec((1, D), lambda i: (0, 0)),
            pl.BlockSpec((1, D), lambda i: (0, 0)),
        ],
        out_specs=pl.BlockSpec((_BLK, D), lambda i: (i, 0)),
        out_shape=jax.ShapeDtypeStruct((N, D), jnp.float32),
    )(acc_parts, acc_parts, y, dinv,
      b.reshape(1, D), ln_gamma.reshape(1, D), ln_beta.reshape(1, D))
    return out


# speedup vs baseline: 27.3806x; 1.0881x over previous
import functools

import jax
import jax.numpy as jnp
from jax import lax
from jax.experimental import pallas as pl
from jax.experimental.pallas import tpu as pltpu
from jax.experimental.pallas import tpu_sc as plsc

N = 10000
E = 320000
D = 128

NC, NS = 2, 16
NW = NC * NS
CHUNK = 64
NCHUNK = 160
EPT = NCHUNK * CHUNK
E_PAD = NW * EPT
ACC_ROWS = 10112
ZPT = ACC_ROWS // NS
Y_ROWS = 10240

_mesh = plsc.VectorSubcoreMesh(core_axis_name="c", subcore_axis_name="s")


@functools.partial(
    pl.kernel,
    out_type=jax.ShapeDtypeStruct((NC * ACC_ROWS,), jnp.float32),
    mesh=_mesh,
    scratch_types=[
        [pltpu.VMEM((8, CHUNK), jnp.int32)] * 4,
        pltpu.VMEM((CHUNK,), jnp.float32),
        pltpu.VMEM((ZPT + 8,), jnp.float32),
        pltpu.VMEM_SHARED((ACC_ROWS,), jnp.float32),
        [pltpu.SemaphoreType.DMA] * 4,
        [pltpu.SemaphoreType.DMA] * 4,
    ],
)
def _deg_kernel(dst_hbm, deg_out, didx, ones_v, stage_v, degs, isems, ssems):
    cid = lax.axis_index("c")
    sid = lax.axis_index("s")
    w = sid * NC + cid
    row0 = sid * ZPT
    zeros16 = jnp.zeros((16,), jnp.float32)
    ones16 = jnp.ones((16,), jnp.float32)
    for j in range(CHUNK // 16):
        ones_v[pl.ds(j * 16, 16)] = ones16

    def zbody(j, carry):
        stage_v[pl.ds(j * 16, 16)] = zeros16
        return carry

    lax.fori_loop(0, (ZPT + 8) // 16, zbody, 0)
    pltpu.sync_copy(stage_v.at[pl.ds(0, ZPT)], degs.at[pl.ds(row0, ZPT)])
    plsc.subcore_barrier()

    NSUP = NCHUNK // 8

    def start_fetch(q, s):
        pltpu.async_copy(dst_hbm.at[w, pl.ds(q * 8, 8)], didx[s], isems[s])

    def wait_fetch(s):
        pltpu.make_async_copy(dst_hbm.at[0, pl.ds(0, 8)], didx[s],
                              isems[s]).wait()

    def scatter_super(s):
        for j in range(8):
            pltpu.async_copy(ones_v, degs.at[didx[s].at[j]], ssems[s],
                             add=True)

    def drain_super(s):
        for j in range(8):
            pltpu.make_async_copy(ones_v, degs.at[didx[s].at[0]],
                                  ssems[s]).wait()

    for s in range(2):
        start_fetch(s, s)
    for q in range(2):
        wait_fetch(q)
        scatter_super(q)
        start_fetch(q + 2, q + 2)

    def group(g, carry):
        q0 = g * 4 + 2
        for k in range(4):
            p = (2 + k) % 4
            pf = k % 4
            wait_fetch(p)
            scatter_super(p)
            drain_super(pf)
            start_fetch(q0 + k + 2, pf)
        return carry

    lax.fori_loop(0, (NSUP - 4) // 4, group, 0)
    for q in range(NSUP - 2, NSUP):
        wait_fetch(q % 4)
        scatter_super(q % 4)
    for s in range(4):
        drain_super(s)
    plsc.subcore_barrier()
    pltpu.sync_copy(degs.at[pl.ds(row0, ZPT)], stage_v.at[pl.ds(0, ZPT)])
    pltpu.sync_copy(stage_v.at[pl.ds(0, ZPT)],
                    deg_out.at[pl.ds(cid * ACC_ROWS + row0, ZPT)])


@functools.partial(
    pl.kernel,
    out_type=jax.ShapeDtypeStruct((NC, ACC_ROWS, D), jnp.float32),
    mesh=_mesh,
    scratch_types=[
        [pltpu.VMEM((CHUNK,), jnp.int32)] * 4,
        [pltpu.VMEM((CHUNK,), jnp.int32)] * 4,
        [pltpu.VMEM((CHUNK, D), jnp.float32)] * 4,
        pltpu.VMEM_SHARED((ACC_ROWS, D), jnp.float32),
        [pltpu.SemaphoreType.DMA] * 4,
        [pltpu.SemaphoreType.DMA] * 4,
        [pltpu.SemaphoreType.DMA] * 4,
    ],
)
def _edge_kernel(y_hbm, src_hbm, dst_hbm, out_hbm,
                 sidx, didx, bufs, acc, gsems, ssems, isems):
    cid = lax.axis_index("c")
    sid = lax.axis_index("s")
    w = sid * NC + cid
    row0 = sid * ZPT
    zeros16 = jnp.zeros((16,), jnp.float32)
    buf0 = bufs[0]

    def zbody(r, carry):
        for j in range(D // 16):
            buf0[r, pl.ds(j * 16, 16)] = zeros16
        return carry

    lax.fori_loop(0, CHUNK, zbody, 0)
    _rem = ZPT % CHUNK
    for k in range(ZPT // CHUNK):
        pltpu.sync_copy(buf0, acc.at[pl.ds(row0 + k * CHUNK, CHUNK)])
    if _rem:
        pltpu.sync_copy(buf0.at[pl.ds(0, _rem)],
                        acc.at[pl.ds(row0 + (ZPT // CHUNK) * CHUNK, _rem)])
    plsc.subcore_barrier()

    def start_fetch(c, s):
        pltpu.async_copy(src_hbm.at[w, c], sidx[s], isems[s])
        pltpu.async_copy(dst_hbm.at[w, c], didx[s], isems[s])

    def wait_fetch(s):
        pltpu.make_async_copy(src_hbm.at[0, 0], sidx[s], isems[s]).wait()
        pltpu.make_async_copy(dst_hbm.at[0, 0], didx[s], isems[s]).wait()

    def start_gather(s):
        pltpu.async_copy(y_hbm.at[sidx[s]], bufs[s], gsems[s])

    def wait_gather(s):
        pltpu.make_async_copy(y_hbm.at[sidx[s]], bufs[s], gsems[s]).wait()

    def start_scatter(s):
        pltpu.async_copy(bufs[s], acc.at[didx[s]], ssems[s], add=True)

    def wait_scatter(s):
        pltpu.make_async_copy(bufs[s], acc.at[didx[s]], ssems[s]).wait()

    for s in range(4):
        start_fetch(s, s)
    wait_fetch(0)
    start_gather(0)
    for c in range(2):
        wait_gather(c)
        start_scatter(c)
        wait_fetch(c + 1)
        start_gather(c + 1)

    def group(g, carry):
        c0 = g * 4 + 2
        for k in range(4):
            c = c0 + k
            b0 = (2 + k) % 4
            b1 = (3 + k) % 4
            b2 = k % 4
            wait_gather(b0)
            start_scatter(b0)
            wait_scatter(b2)
            start_fetch(c + 2, b2)
            wait_fetch(b1)
            start_gather(b1)
        return carry

    lax.fori_loop(0, (NCHUNK - 4) // 4, group, 0)
    wait_gather(2)
    start_scatter(2)
    wait_scatter(0)
    wait_fetch(3)
    start_gather(3)
    wait_gather(3)
    start_scatter(3)
    wait_scatter(1)
    wait_scatter(2)
    wait_scatter(3)
    plsc.subcore_barrier()
    for k in range(ZPT // CHUNK):
        pltpu.sync_copy(acc.at[pl.ds(row0 + k * CHUNK, CHUNK)], buf0)
        pltpu.sync_copy(buf0, out_hbm.at[cid, pl.ds(row0 + k * CHUNK, CHUNK)])
    if _rem:
        _off = row0 + (ZPT // CHUNK) * CHUNK
        pltpu.sync_copy(acc.at[pl.ds(_off, _rem)], buf0.at[pl.ds(0, _rem)])
        pltpu.sync_copy(buf0.at[pl.ds(0, _rem)],
                        out_hbm.at[cid, pl.ds(_off, _rem)])


_BLK = 1000


def _mm_body(x_ref, w_ref, d0_ref, d1_ref, y_ref, dinv_ref):
    deg = d0_ref[...] + d1_ref[...] + 1.0
    dinv = lax.rsqrt(deg)
    xw = jnp.dot(x_ref[...], w_ref[...], preferred_element_type=jnp.float32)
    y_ref[...] = xw * dinv
    dinv_ref[...] = dinv


def _fin_body(a0_ref, a1_ref, y_ref, dinv_ref, b_ref, g_ref, be_ref, o_ref):
    s = a0_ref[0] + a1_ref[0] + y_ref[...]
    pre = s * dinv_ref[...] + b_ref[...]
    mu = jnp.mean(pre, axis=-1, keepdims=True)
    ctr = pre - mu
    var = jnp.mean(ctr * ctr, axis=-1, keepdims=True)
    h = ctr * lax.rsqrt(var + 1e-5) * g_ref[...] + be_ref[...]
    o_ref[...] = jnp.maximum(h, 0.0)


def kernel(x, edge_index, W, b, ln_gamma, ln_beta):
    ei = edge_index.astype(jnp.int32)
    pad = N + (jnp.arange(E_PAD - E, dtype=jnp.int32) % 8)
    src_p = jnp.concatenate([ei[0], pad]).reshape(NW, NCHUNK, CHUNK)
    dst_p = jnp.concatenate([ei[1], pad]).reshape(NW, NCHUNK, CHUNK)

    deg_parts = _deg_kernel(dst_p)
    d0 = deg_parts[:N].reshape(N, 1)
    d1 = deg_parts[ACC_ROWS:ACC_ROWS + N].reshape(N, 1)

    _YB = Y_ROWS // 10
    y, dinv = pl.pallas_call(
        _mm_body,
        grid=(10,),
        in_specs=[
            pl.BlockSpec((_YB, D), lambda i: (i, 0)),
            pl.BlockSpec((D, D), lambda i: (0, 0)),
            pl.BlockSpec((_YB, 1), lambda i: (i, 0)),
            pl.BlockSpec((_YB, 1), lambda i: (i, 0)),
        ],
        out_specs=[
            pl.BlockSpec((_YB, D), lambda i: (i, 0)),
            pl.BlockSpec((_YB, 1), lambda i: (i, 0)),
        ],
        out_shape=[
            jax.ShapeDtypeStruct((Y_ROWS, D), jnp.float32),
            jax.ShapeDtypeStruct((Y_ROWS, 1), jnp.float32),
        ],
    )(x, W, d0, d1)

    acc_parts = _edge_kernel(y, src_p, dst_p)

    out = pl.pallas_call(
        _fin_body,
        grid=(N // _BLK,),
        in_specs=[
            pl.BlockSpec((1, _BLK, D), lambda i: (0, i, 0)),
            pl.BlockSpec((1, _BLK, D), lambda i: (1, i, 0)),
            pl.BlockSpec((_BLK, D), lambda i: (i, 0)),
            pl.BlockSpec((_BLK, 1), lambda i: (i, 0)),
            pl.BlockSpec((1, D), lambda i: (0, 0)),
            pl.BlockSpec((1, D), lambda i: (0, 0)),
            pl.BlockSpec((1, D), lambda i: (0, 0)),
        ],
        out_specs=pl.BlockSpec((_BLK, D), lambda i: (i, 0)),
        out_shape=jax.ShapeDtypeStruct((N, D), jnp.float32),
    )(acc_parts, acc_parts, y, dinv,
      b.reshape(1, D), ln_gamma.reshape(1, D), ln_beta.reshape(1, D))
    return out
